# baseline (speedup 1.0000x reference)
import dataclasses

import jax
import jax.numpy as jnp
from jax import lax
from jax.experimental import pallas as pl
from jax.experimental.pallas import tpu as pltpu
from jax.experimental.pallas import tpu_sc as plsc

_MARGIN = 2.0
_B = 16384
_HID = 32
_NW = 32
_PER_W = _B // _NW
_CHUNK = 64
_NCHUNK = _PER_W // _CHUNK
_ENT = 1000000
_NCOL = _ENT // 128
_TAIL = _NCOL * 128
_ENT_ROWS = _ENT // 4
_REL_ROWS = 250
_SWEEP_MAIN = 244


def _make_cp():
    cp = pltpu.CompilerParams(use_tc_tiling_on_sc=True)
    if "needs_layout_passes" in pltpu.CompilerParams.__dataclass_fields__:
        cp = dataclasses.replace(cp, needs_layout_passes=False)
    return cp


def _detile_body(ent_hbm, aux_hbm, out_hbm,
                 vin0, vin1, vin2, vin3, vout0, vout1, vout2, vout3, vaux,
                 rd0, rd1, rd2, rd3, wr0, wr1, wr2, wr3):
    wid = lax.axis_index("s") * 2 + lax.axis_index("c")
    vin = (vin0, vin1, vin2, vin3)
    vout = (vout0, vout1, vout2, vout3)
    rds = (rd0, rd1, rd2, rd3)
    wrs = (wr0, wr1, wr2, wr3)

    def col(t):
        return (t * 32 + wid) * 128

    def issue_read(t, p):
        pltpu.async_copy(ent_hbm.at[:, pl.ds(col(t), 128)],
                         vin[p].at[:, pl.ds(0, 128)], rds[p])

    def drain_read(p):
        pltpu.make_async_copy(ent_hbm.at[:, pl.ds(0, 128)],
                              vin[p].at[:, pl.ds(0, 128)], rds[p]).wait()

    iot = lax.iota(jnp.int32, 16)
    hh0 = iot
    hh1 = jnp.full((16,), 16, jnp.int32) + iot

    def transform(p):
        for q in range(32):
            for k in range(8):
                hh = hh0 if k % 2 == 0 else hh1
                ww = jnp.full((16,), 4 * q + k // 2, jnp.int32)
                vout[p][q, pl.ds(16 * k, 16)] = plsc.load_gather(
                    vin[p], [hh, ww])

    def issue_write(t, p):
        pltpu.async_copy(vout[p], out_hbm.at[pl.ds((t * 32 + wid) * 32, 32)],
                         wrs[p])

    def drain_write(p):
        pltpu.make_async_copy(vout[p], out_hbm.at[pl.ds(0, 32)],
                              wrs[p]).wait()

    issue_read(0, 0)
    issue_read(1, 1)
    issue_read(2, 2)

    @pl.loop(0, _SWEEP_MAIN, step=4)
    def _(t):
        for sp in range(4):
            u = t + sp

            @pl.when(u + 3 < _SWEEP_MAIN)
            def _():
                issue_read(u + 3, (sp + 3) % 4)

            drain_read(sp)

            @pl.when(u >= 4)
            def _():
                drain_write(sp)

            transform(sp)
            issue_write(u, sp)

    drain_write(0)
    drain_write(1)
    drain_write(2)
    drain_write(3)

    @pl.when(wid < _NCOL - _SWEEP_MAIN * 32)
    def _():
        c = _SWEEP_MAIN * 32 + wid
        pltpu.sync_copy(ent_hbm.at[:, pl.ds(c * 128, 128)],
                        vin0.at[:, pl.ds(0, 128)])
        transform(0)
        pltpu.sync_copy(vout0, out_hbm.at[pl.ds(c * 32, 32)])

    @pl.when(wid == 0)
    def _():
        pltpu.sync_copy(aux_hbm, vaux)
        pltpu.sync_copy(vaux, out_hbm.at[pl.ds(_ENT_ROWS - 16, 16)])


@jax.jit
def _detile(ent_t, aux):
    mesh = plsc.VectorSubcoreMesh(core_axis_name="c", subcore_axis_name="s")
    run = pl.kernel(
        _detile_body,
        out_type=jax.ShapeDtypeStruct((_ENT_ROWS, 128), jnp.float32),
        mesh=mesh,
        compiler_params=_make_cp(),
        scratch_types=(
            [pltpu.VMEM((_HID, 133), jnp.float32)] * 4
            + [pltpu.VMEM((32, 128), jnp.float32)] * 4
            + [pltpu.VMEM((16, 128), jnp.float32)]
            + [pltpu.SemaphoreType.DMA] * 8
        ),
    )
    return run(ent_t, aux)


def _tec_body(ph_hbm, pt_hbm, pr_hbm, nh_hbm, nt_hbm, nr_hbm,
              ent_hbm, rel_hbm, nrm_hbm, out_hbm,
              iph, ipt, ipr, inh, int_, inr,
              gph, gpt, gnh, gnt,
              bph, bpt, bnh, bnt,
              trel, tnrm,
              stage, sem):
    wid = lax.axis_index("s") * 2 + lax.axis_index("c")
    base = wid * _PER_W

    crel = pltpu.async_copy(rel_hbm, trel, sem)
    cnrm = pltpu.async_copy(nrm_hbm, tnrm, sem)

    pltpu.sync_copy(ph_hbm.at[pl.ds(base, _PER_W)], iph)
    pltpu.sync_copy(pt_hbm.at[pl.ds(base, _PER_W)], ipt)
    pltpu.sync_copy(pr_hbm.at[pl.ds(base, _PER_W)], ipr)
    pltpu.sync_copy(nh_hbm.at[pl.ds(base, _PER_W)], inh)
    pltpu.sync_copy(nr_hbm.at[pl.ds(base, _PER_W)], inr)
    pltpu.sync_copy(nt_hbm.at[pl.ds(base, _PER_W)], int_)

    @pl.loop(0, _PER_W, step=16)
    def _(j):
        sl = pl.ds(j, 16)
        gph[sl] = jax.lax.shift_right_logical(iph[sl], 2)
        gpt[sl] = jax.lax.shift_right_logical(ipt[sl], 2)
        gnh[sl] = jax.lax.shift_right_logical(inh[sl], 2)
        gnt[sl] = jax.lax.shift_right_logical(int_[sl], 2)

    crel.wait()
    cnrm.wait()

    acc = jnp.float32(0.0)
    for k in range(_NCHUNK):
        sl = pl.ds(k * _CHUNK, _CHUNK)
        cps = [
            pltpu.async_copy(ent_hbm.at[gph.at[sl]], bph, sem),
            pltpu.async_copy(ent_hbm.at[gpt.at[sl]], bpt, sem),
            pltpu.async_copy(ent_hbm.at[gnh.at[sl]], bnh, sem),
            pltpu.async_copy(ent_hbm.at[gnt.at[sl]], bnt, sem),
        ]
        for c in cps:
            c.wait()

        def group_rows(g, a):
            goff = g * 16
            vph = iph[pl.ds(k * _CHUNK + goff, 16)]
            vpt = ipt[pl.ds(k * _CHUNK + goff, 16)]
            vpr = ipr[pl.ds(k * _CHUNK + goff, 16)]
            vnh = inh[pl.ds(k * _CHUNK + goff, 16)]
            vnt = int_[pl.ds(k * _CHUNK + goff, 16)]
            vnr = inr[pl.ds(k * _CHUNK + goff, 16)]
            for j in range(16):
                i = goff + j
                cph = (vph[j] & 3) * 32
                cpt = (vpt[j] & 3) * 32
                cnh = (vnh[j] & 3) * 32
                cnt = (vnt[j] & 3) * 32
                r_p = vpr[j]
                r_n = vnr[j]
                rp_row = jax.lax.shift_right_logical(r_p, 2)
                rp_col = (r_p & 3) * 32
                rn_row = jax.lax.shift_right_logical(r_n, 2)
                rn_col = (r_n & 3) * 32

                ph0 = bph[i, pl.ds(cph, 16)]
                ph1 = bph[i, pl.ds(cph + 16, 16)]
                pt0 = bpt[i, pl.ds(cpt, 16)]
                pt1 = bpt[i, pl.ds(cpt + 16, 16)]
                nh0 = bnh[i, pl.ds(cnh, 16)]
                nh1 = bnh[i, pl.ds(cnh + 16, 16)]
                nt0 = bnt[i, pl.ds(cnt, 16)]
                nt1 = bnt[i, pl.ds(cnt + 16, 16)]
                pr0 = trel[rp_row, pl.ds(rp_col, 16)]
                pr1 = trel[rp_row, pl.ds(rp_col + 16, 16)]
                pn0 = tnrm[rp_row, pl.ds(rp_col, 16)]
                pn1 = tnrm[rp_row, pl.ds(rp_col + 16, 16)]
                nr0 = trel[rn_row, pl.ds(rn_col, 16)]
                nr1 = trel[rn_row, pl.ds(rn_col + 16, 16)]
                nn0 = tnrm[rn_row, pl.ds(rn_col, 16)]
                nn1 = tnrm[rn_row, pl.ds(rn_col + 16, 16)]

                pd0 = ph0 - pt0
                pd1 = ph1 - pt1
                pdot = jnp.sum(pd0 * pn0 + pd1 * pn1)
                ps0 = pd0 + pr0 - pdot * pn0
                ps1 = pd1 + pr1 - pdot * pn1
                p_score = jnp.sum(jnp.abs(ps0) + jnp.abs(ps1))

                nd0 = nh0 - nt0
                nd1 = nh1 - nt1
                ndot = jnp.sum(nd0 * nn0 + nd1 * nn1)
                ns0 = nd0 + nr0 - ndot * nn0
                ns1 = nd1 + nr1 - ndot * nn1
                n_score = jnp.sum(jnp.abs(ns0) + jnp.abs(ns1))

                a = a + jnp.maximum(p_score - n_score + _MARGIN, 0.0)
            return a

        acc = lax.fori_loop(0, _CHUNK // 16, group_rows, acc)

    lane = lax.iota(jnp.int32, 16)
    stage[...] = jnp.where(lane == 0, acc, jnp.float32(0.0))
    pltpu.sync_copy(stage, out_hbm.at[wid])


@jax.jit
def _transh_loss_partials(p_h, p_t, p_r, n_h, n_t, n_r,
                          ent4, rel4, nrm4):
    mesh = plsc.VectorSubcoreMesh(core_axis_name="c", subcore_axis_name="s")
    run = pl.kernel(
        _tec_body,
        out_type=jax.ShapeDtypeStruct((_NW, 16), jnp.float32),
        mesh=mesh,
        compiler_params=_make_cp(),
        scratch_types=[
            pltpu.VMEM((_PER_W,), jnp.int32),
            pltpu.VMEM((_PER_W,), jnp.int32),
            pltpu.VMEM((_PER_W,), jnp.int32),
            pltpu.VMEM((_PER_W,), jnp.int32),
            pltpu.VMEM((_PER_W,), jnp.int32),
            pltpu.VMEM((_PER_W,), jnp.int32),
            pltpu.VMEM((_PER_W,), jnp.int32),
            pltpu.VMEM((_PER_W,), jnp.int32),
            pltpu.VMEM((_PER_W,), jnp.int32),
            pltpu.VMEM((_PER_W,), jnp.int32),
            pltpu.VMEM((_CHUNK, 128), jnp.float32),
            pltpu.VMEM((_CHUNK, 128), jnp.float32),
            pltpu.VMEM((_CHUNK, 128), jnp.float32),
            pltpu.VMEM((_CHUNK, 128), jnp.float32),
            pltpu.VMEM((_REL_ROWS, 128), jnp.float32),
            pltpu.VMEM((_REL_ROWS, 128), jnp.float32),
            pltpu.VMEM((16,), jnp.float32),
            pltpu.SemaphoreType.DMA,
        ],
    )
    return run(p_h.astype(jnp.int32), p_t.astype(jnp.int32),
               p_r.astype(jnp.int32), n_h.astype(jnp.int32),
               n_t.astype(jnp.int32), n_r.astype(jnp.int32),
               ent4, rel4, nrm4)


def kernel(p_h, p_t, p_r, n_h, n_t, n_r, ent_emb, rel_emb, norm_vec):
    aux = ent_emb[_TAIL:, :].reshape(16, 128)
    ent4 = _detile(ent_emb.T, aux)

    rel4 = rel_emb.reshape(_REL_ROWS, 128)
    nrm4 = norm_vec.reshape(_REL_ROWS, 128)
    partials = _transh_loss_partials(p_h, p_t, p_r, n_h, n_t, n_r,
                                     ent4, rel4, nrm4)
    return jnp.sum(partials)


# speedup vs baseline: 2.0394x; 2.0394x over previous
import dataclasses

import jax
import jax.numpy as jnp
from jax import lax
from jax.experimental import pallas as pl
from jax.experimental.pallas import tpu as pltpu
from jax.experimental.pallas import tpu_sc as plsc

_MARGIN = 2.0
_B = 16384
_HID = 32
_NW = 32
_PER_W = _B // _NW
_CHUNK = 64
_NCHUNK = _PER_W // _CHUNK
_ENT = 1000000
_NCOL = _ENT // 128
_TAIL = _NCOL * 128
_ENT_ROWS = _ENT // 4
_REL_ROWS = 250
_SWEEP_MAIN = 244


def _make_cp():
    cp = pltpu.CompilerParams(use_tc_tiling_on_sc=True)
    if "needs_layout_passes" in pltpu.CompilerParams.__dataclass_fields__:
        cp = dataclasses.replace(cp, needs_layout_passes=False)
    return cp


def _tc_detile_body(x_ref, o_ref):
    y = x_ref[...].T.reshape(1024, 4, 32)
    o_ref[...] = jnp.concatenate([y[:, d, :] for d in range(4)], axis=1)


@jax.jit
def _detile(ent_t):
    return pl.pallas_call(
        _tc_detile_body,
        grid=(245,),
        in_specs=[pl.BlockSpec((_HID, 4096), lambda i: (0, i))],
        out_specs=pl.BlockSpec((1024, 128), lambda i: (i, 0)),
        out_shape=jax.ShapeDtypeStruct((_ENT_ROWS, 128), jnp.float32),
    )(ent_t)


def _tec_body(ph_hbm, pt_hbm, pr_hbm, nh_hbm, nt_hbm, nr_hbm,
              ent_hbm, rel_hbm, nrm_hbm, out_hbm,
              iph, ipt, ipr, inh, int_, inr,
              gph, gpt, gnh, gnt,
              bph, bpt, bnh, bnt,
              trel, tnrm,
              stage, sem):
    wid = lax.axis_index("s") * 2 + lax.axis_index("c")
    base = wid * _PER_W

    crel = pltpu.async_copy(rel_hbm, trel, sem)
    cnrm = pltpu.async_copy(nrm_hbm, tnrm, sem)

    pltpu.sync_copy(ph_hbm.at[pl.ds(base, _PER_W)], iph)
    pltpu.sync_copy(pt_hbm.at[pl.ds(base, _PER_W)], ipt)
    pltpu.sync_copy(pr_hbm.at[pl.ds(base, _PER_W)], ipr)
    pltpu.sync_copy(nh_hbm.at[pl.ds(base, _PER_W)], inh)
    pltpu.sync_copy(nr_hbm.at[pl.ds(base, _PER_W)], inr)
    pltpu.sync_copy(nt_hbm.at[pl.ds(base, _PER_W)], int_)

    @pl.loop(0, _PER_W, step=16)
    def _(j):
        sl = pl.ds(j, 16)
        gph[sl] = jax.lax.shift_right_logical(iph[sl], 2)
        gpt[sl] = jax.lax.shift_right_logical(ipt[sl], 2)
        gnh[sl] = jax.lax.shift_right_logical(inh[sl], 2)
        gnt[sl] = jax.lax.shift_right_logical(int_[sl], 2)

    crel.wait()
    cnrm.wait()

    acc = jnp.float32(0.0)
    for k in range(_NCHUNK):
        sl = pl.ds(k * _CHUNK, _CHUNK)
        cps = [
            pltpu.async_copy(ent_hbm.at[gph.at[sl]], bph, sem),
            pltpu.async_copy(ent_hbm.at[gpt.at[sl]], bpt, sem),
            pltpu.async_copy(ent_hbm.at[gnh.at[sl]], bnh, sem),
            pltpu.async_copy(ent_hbm.at[gnt.at[sl]], bnt, sem),
        ]
        for c in cps:
            c.wait()

        def group_rows(g, a):
            goff = g * 16
            vph = iph[pl.ds(k * _CHUNK + goff, 16)]
            vpt = ipt[pl.ds(k * _CHUNK + goff, 16)]
            vpr = ipr[pl.ds(k * _CHUNK + goff, 16)]
            vnh = inh[pl.ds(k * _CHUNK + goff, 16)]
            vnt = int_[pl.ds(k * _CHUNK + goff, 16)]
            vnr = inr[pl.ds(k * _CHUNK + goff, 16)]
            for j in range(16):
                i = goff + j
                cph = (vph[j] & 3) * 32
                cpt = (vpt[j] & 3) * 32
                cnh = (vnh[j] & 3) * 32
                cnt = (vnt[j] & 3) * 32
                r_p = vpr[j]
                r_n = vnr[j]
                rp_row = jax.lax.shift_right_logical(r_p, 2)
                rp_col = (r_p & 3) * 32
                rn_row = jax.lax.shift_right_logical(r_n, 2)
                rn_col = (r_n & 3) * 32

                ph0 = bph[i, pl.ds(cph, 16)]
                ph1 = bph[i, pl.ds(cph + 16, 16)]
                pt0 = bpt[i, pl.ds(cpt, 16)]
                pt1 = bpt[i, pl.ds(cpt + 16, 16)]
                nh0 = bnh[i, pl.ds(cnh, 16)]
                nh1 = bnh[i, pl.ds(cnh + 16, 16)]
                nt0 = bnt[i, pl.ds(cnt, 16)]
                nt1 = bnt[i, pl.ds(cnt + 16, 16)]
                pr0 = trel[rp_row, pl.ds(rp_col, 16)]
                pr1 = trel[rp_row, pl.ds(rp_col + 16, 16)]
                pn0 = tnrm[rp_row, pl.ds(rp_col, 16)]
                pn1 = tnrm[rp_row, pl.ds(rp_col + 16, 16)]
                nr0 = trel[rn_row, pl.ds(rn_col, 16)]
                nr1 = trel[rn_row, pl.ds(rn_col + 16, 16)]
                nn0 = tnrm[rn_row, pl.ds(rn_col, 16)]
                nn1 = tnrm[rn_row, pl.ds(rn_col + 16, 16)]

                pd0 = ph0 - pt0
                pd1 = ph1 - pt1
                pdot = jnp.sum(pd0 * pn0 + pd1 * pn1)
                ps0 = pd0 + pr0 - pdot * pn0
                ps1 = pd1 + pr1 - pdot * pn1
                p_score = jnp.sum(jnp.abs(ps0) + jnp.abs(ps1))

                nd0 = nh0 - nt0
                nd1 = nh1 - nt1
                ndot = jnp.sum(nd0 * nn0 + nd1 * nn1)
                ns0 = nd0 + nr0 - ndot * nn0
                ns1 = nd1 + nr1 - ndot * nn1
                n_score = jnp.sum(jnp.abs(ns0) + jnp.abs(ns1))

                a = a + jnp.maximum(p_score - n_score + _MARGIN, 0.0)
            return a

        acc = lax.fori_loop(0, _CHUNK // 16, group_rows, acc)

    lane = lax.iota(jnp.int32, 16)
    stage[...] = jnp.where(lane == 0, acc, jnp.float32(0.0))
    pltpu.sync_copy(stage, out_hbm.at[wid])


@jax.jit
def _transh_loss_partials(p_h, p_t, p_r, n_h, n_t, n_r,
                          ent4, rel4, nrm4):
    mesh = plsc.VectorSubcoreMesh(core_axis_name="c", subcore_axis_name="s")
    run = pl.kernel(
        _tec_body,
        out_type=jax.ShapeDtypeStruct((_NW, 16), jnp.float32),
        mesh=mesh,
        compiler_params=_make_cp(),
        scratch_types=[
            pltpu.VMEM((_PER_W,), jnp.int32),
            pltpu.VMEM((_PER_W,), jnp.int32),
            pltpu.VMEM((_PER_W,), jnp.int32),
            pltpu.VMEM((_PER_W,), jnp.int32),
            pltpu.VMEM((_PER_W,), jnp.int32),
            pltpu.VMEM((_PER_W,), jnp.int32),
            pltpu.VMEM((_PER_W,), jnp.int32),
            pltpu.VMEM((_PER_W,), jnp.int32),
            pltpu.VMEM((_PER_W,), jnp.int32),
            pltpu.VMEM((_PER_W,), jnp.int32),
            pltpu.VMEM((_CHUNK, 128), jnp.float32),
            pltpu.VMEM((_CHUNK, 128), jnp.float32),
            pltpu.VMEM((_CHUNK, 128), jnp.float32),
            pltpu.VMEM((_CHUNK, 128), jnp.float32),
            pltpu.VMEM((_REL_ROWS, 128), jnp.float32),
            pltpu.VMEM((_REL_ROWS, 128), jnp.float32),
            pltpu.VMEM((16,), jnp.float32),
            pltpu.SemaphoreType.DMA,
        ],
    )
    return run(p_h.astype(jnp.int32), p_t.astype(jnp.int32),
               p_r.astype(jnp.int32), n_h.astype(jnp.int32),
               n_t.astype(jnp.int32), n_r.astype(jnp.int32),
               ent4, rel4, nrm4)


def kernel(p_h, p_t, p_r, n_h, n_t, n_r, ent_emb, rel_emb, norm_vec):
    ent4 = _detile(ent_emb.T)

    rel4 = rel_emb.reshape(_REL_ROWS, 128)
    nrm4 = norm_vec.reshape(_REL_ROWS, 128)
    partials = _transh_loss_partials(p_h, p_t, p_r, n_h, n_t, n_r,
                                     ent4, rel4, nrm4)
    return jnp.sum(partials)


# speedup vs baseline: 2.0426x; 1.0016x over previous
import dataclasses

import jax
import jax.numpy as jnp
from jax import lax
from jax.experimental import pallas as pl
from jax.experimental.pallas import tpu as pltpu
from jax.experimental.pallas import tpu_sc as plsc

_MARGIN = 2.0
_B = 16384
_HID = 32
_NW = 32
_PER_W = _B // _NW
_CHUNK = 64
_NCHUNK = _PER_W // _CHUNK
_ENT = 1000000
_ENT_ROWS = _ENT // 4
_REL_ROWS = 250


def _make_cp():
    cp = pltpu.CompilerParams(use_tc_tiling_on_sc=True)
    if "needs_layout_passes" in pltpu.CompilerParams.__dataclass_fields__:
        cp = dataclasses.replace(cp, needs_layout_passes=False)
    return cp


def _tc_detile_body(x_ref, o_ref):
    y = x_ref[...].T.reshape(1024, 4, 32)
    o_ref[...] = jnp.concatenate([y[:, d, :] for d in range(4)], axis=1)


@jax.jit
def _detile(ent_t):
    return pl.pallas_call(
        _tc_detile_body,
        grid=(245,),
        in_specs=[pl.BlockSpec((_HID, 4096), lambda i: (0, i))],
        out_specs=pl.BlockSpec((1024, 128), lambda i: (i, 0)),
        out_shape=jax.ShapeDtypeStruct((_ENT_ROWS, 128), jnp.float32),
    )(ent_t)


def _tec_body(ph_hbm, pt_hbm, pr_hbm, nh_hbm, nt_hbm, nr_hbm,
              ent_hbm, rel_hbm, nrm_hbm, out_hbm,
              iph, ipt, ipr, inh, int_, inr,
              gph, gpt, gnh, gnt,
              bph, bpt, bnh, bnt,
              trel, tnrm,
              stage, sem):
    wid = lax.axis_index("s") * 2 + lax.axis_index("c")
    base = wid * _PER_W

    crel = pltpu.async_copy(rel_hbm, trel, sem)
    cnrm = pltpu.async_copy(nrm_hbm, tnrm, sem)

    pltpu.sync_copy(ph_hbm.at[pl.ds(base, _PER_W)], iph)
    pltpu.sync_copy(pt_hbm.at[pl.ds(base, _PER_W)], ipt)
    pltpu.sync_copy(pr_hbm.at[pl.ds(base, _PER_W)], ipr)
    pltpu.sync_copy(nh_hbm.at[pl.ds(base, _PER_W)], inh)
    pltpu.sync_copy(nr_hbm.at[pl.ds(base, _PER_W)], inr)
    pltpu.sync_copy(nt_hbm.at[pl.ds(base, _PER_W)], int_)

    @pl.loop(0, _PER_W, step=16)
    def _(j):
        sl = pl.ds(j, 16)
        gph[sl] = jax.lax.shift_right_logical(iph[sl], 2)
        gpt[sl] = jax.lax.shift_right_logical(ipt[sl], 2)
        gnh[sl] = jax.lax.shift_right_logical(inh[sl], 2)
        gnt[sl] = jax.lax.shift_right_logical(int_[sl], 2)

    crel.wait()
    cnrm.wait()

    acc = jnp.float32(0.0)
    for k in range(_NCHUNK):
        sl = pl.ds(k * _CHUNK, _CHUNK)
        cps = [
            pltpu.async_copy(ent_hbm.at[gph.at[sl]], bph, sem),
            pltpu.async_copy(ent_hbm.at[gpt.at[sl]], bpt, sem),
            pltpu.async_copy(ent_hbm.at[gnh.at[sl]], bnh, sem),
            pltpu.async_copy(ent_hbm.at[gnt.at[sl]], bnt, sem),
        ]
        for c in cps:
            c.wait()

        def group_rows(g, a):
            goff = g * 16
            vph = iph[pl.ds(k * _CHUNK + goff, 16)]
            vpt = ipt[pl.ds(k * _CHUNK + goff, 16)]
            vpr = ipr[pl.ds(k * _CHUNK + goff, 16)]
            vnh = inh[pl.ds(k * _CHUNK + goff, 16)]
            vnt = int_[pl.ds(k * _CHUNK + goff, 16)]
            vnr = inr[pl.ds(k * _CHUNK + goff, 16)]
            for j in range(16):
                i = goff + j
                cph = (vph[j] & 3) * 32
                cpt = (vpt[j] & 3) * 32
                cnh = (vnh[j] & 3) * 32
                cnt = (vnt[j] & 3) * 32
                r_p = vpr[j]
                r_n = vnr[j]
                rp_row = jax.lax.shift_right_logical(r_p, 2)
                rp_col = (r_p & 3) * 32
                rn_row = jax.lax.shift_right_logical(r_n, 2)
                rn_col = (r_n & 3) * 32

                ph0 = bph[i, pl.ds(cph, 16)]
                ph1 = bph[i, pl.ds(cph + 16, 16)]
                pt0 = bpt[i, pl.ds(cpt, 16)]
                pt1 = bpt[i, pl.ds(cpt + 16, 16)]
                nh0 = bnh[i, pl.ds(cnh, 16)]
                nh1 = bnh[i, pl.ds(cnh + 16, 16)]
                nt0 = bnt[i, pl.ds(cnt, 16)]
                nt1 = bnt[i, pl.ds(cnt + 16, 16)]
                pr0 = trel[rp_row, pl.ds(rp_col, 16)]
                pr1 = trel[rp_row, pl.ds(rp_col + 16, 16)]
                pn0 = tnrm[rp_row, pl.ds(rp_col, 16)]
                pn1 = tnrm[rp_row, pl.ds(rp_col + 16, 16)]
                nr0 = trel[rn_row, pl.ds(rn_col, 16)]
                nr1 = trel[rn_row, pl.ds(rn_col + 16, 16)]
                nn0 = tnrm[rn_row, pl.ds(rn_col, 16)]
                nn1 = tnrm[rn_row, pl.ds(rn_col + 16, 16)]

                pd0 = ph0 - pt0
                pd1 = ph1 - pt1
                pdot = jnp.sum(pd0 * pn0 + pd1 * pn1)
                ps0 = pd0 + pr0 - pdot * pn0
                ps1 = pd1 + pr1 - pdot * pn1
                p_score = jnp.sum(jnp.abs(ps0) + jnp.abs(ps1))

                nd0 = nh0 - nt0
                nd1 = nh1 - nt1
                ndot = jnp.sum(nd0 * nn0 + nd1 * nn1)
                ns0 = nd0 + nr0 - ndot * nn0
                ns1 = nd1 + nr1 - ndot * nn1
                n_score = jnp.sum(jnp.abs(ns0) + jnp.abs(ns1))

                a = a + jnp.maximum(p_score - n_score + _MARGIN, 0.0)
            return a

        acc = lax.fori_loop(0, _CHUNK // 16, group_rows, acc)

    lane = lax.iota(jnp.int32, 16)
    stage[...] = jnp.where(lane == 0, acc, jnp.float32(0.0))
    pltpu.sync_copy(stage, out_hbm.at[wid])


@jax.jit
def _transh_loss_partials(p_h, p_t, p_r, n_h, n_t, n_r,
                          ent4, rel4, nrm4):
    mesh = plsc.VectorSubcoreMesh(core_axis_name="c", subcore_axis_name="s")
    run = pl.kernel(
        _tec_body,
        out_type=jax.ShapeDtypeStruct((_NW, 16), jnp.float32),
        mesh=mesh,
        compiler_params=_make_cp(),
        scratch_types=[
            pltpu.VMEM((_PER_W,), jnp.int32),
            pltpu.VMEM((_PER_W,), jnp.int32),
            pltpu.VMEM((_PER_W,), jnp.int32),
            pltpu.VMEM((_PER_W,), jnp.int32),
            pltpu.VMEM((_PER_W,), jnp.int32),
            pltpu.VMEM((_PER_W,), jnp.int32),
            pltpu.VMEM((_PER_W,), jnp.int32),
            pltpu.VMEM((_PER_W,), jnp.int32),
            pltpu.VMEM((_PER_W,), jnp.int32),
            pltpu.VMEM((_PER_W,), jnp.int32),
            pltpu.VMEM((_CHUNK, 128), jnp.float32),
            pltpu.VMEM((_CHUNK, 128), jnp.float32),
            pltpu.VMEM((_CHUNK, 128), jnp.float32),
            pltpu.VMEM((_CHUNK, 128), jnp.float32),
            pltpu.VMEM((_REL_ROWS, 128), jnp.float32),
            pltpu.VMEM((_REL_ROWS, 128), jnp.float32),
            pltpu.VMEM((16,), jnp.float32),
            pltpu.SemaphoreType.DMA,
        ],
    )
    return run(p_h.astype(jnp.int32), p_t.astype(jnp.int32),
               p_r.astype(jnp.int32), n_h.astype(jnp.int32),
               n_t.astype(jnp.int32), n_r.astype(jnp.int32),
               ent4, rel4, nrm4)


def kernel(p_h, p_t, p_r, n_h, n_t, n_r, ent_emb, rel_emb, norm_vec):
    ent4 = _detile(ent_emb.T)

    rel4 = rel_emb.reshape(_REL_ROWS, 128)
    nrm4 = norm_vec.reshape(_REL_ROWS, 128)
    partials = _transh_loss_partials(p_h, p_t, p_r, n_h, n_t, n_r,
                                     ent4, rel4, nrm4)
    return jnp.sum(partials)
